# baseline (speedup 1.0000x reference)
import functools

import jax
import jax.numpy as jnp
from jax import lax
from jax.experimental import pallas as pl
from jax.experimental.pallas import tpu as pltpu
from jax.experimental.pallas import tpu_sc as plsc

_NC = 2
_NS = 16
_NW = _NC * _NS
_L = 16


def _mesh():
    return plsc.VectorSubcoreMesh(core_axis_name="c", subcore_axis_name="s")


def _wid():
    return lax.axis_index("s") * _NC + lax.axis_index("c")


def _make_idx_kernel(N, res):
    C = 16384
    per = N // _NW
    hi = jnp.float32(res - 1.001)
    scale = jnp.float32(res - 1)

    @functools.partial(
        pl.kernel,
        mesh=_mesh(),
        compiler_params=pltpu.CompilerParams(needs_layout_passes=False),
        out_type=jax.ShapeDtypeStruct((N,), jnp.int32),
        scratch_types=[
            pltpu.VMEM((C,), jnp.float32),
            pltpu.VMEM((C,), jnp.float32),
            pltpu.VMEM((C,), jnp.float32),
            pltpu.VMEM((C,), jnp.int32),
        ],
    )
    def k(x_hbm, y_hbm, z_hbm, idx_hbm, xb, yb, zb, ob):
        base = _wid() * per

        def chunk(ci, carry):
            off = base + ci * C
            pltpu.sync_copy(x_hbm.at[pl.ds(off, C)], xb)
            pltpu.sync_copy(y_hbm.at[pl.ds(off, C)], yb)
            pltpu.sync_copy(z_hbm.at[pl.ds(off, C)], zb)

            def vreg(i, c2):
                s = pl.ds(i * _L, _L)
                ix = jnp.clip(xb[s] * scale, 0.0, hi).astype(jnp.int32)
                iy = jnp.clip(yb[s] * scale, 0.0, hi).astype(jnp.int32)
                iz = jnp.clip(zb[s] * scale, 0.0, hi).astype(jnp.int32)
                ob[s] = ix * (res * res) + iy * res + iz
                return c2

            lax.fori_loop(0, C // _L, vreg, 0)
            pltpu.sync_copy(ob, idx_hbm.at[pl.ds(off, C)])
            return carry

        lax.fori_loop(0, per // C, chunk, 0)

    return k


def _make_scatter_kernel(N, NV):
    C = 8192
    NCH = N // C
    per_slab = NV // _NW

    OVF = C + _L

    @functools.partial(
        pl.kernel,
        mesh=_mesh(),
        compiler_params=pltpu.CompilerParams(needs_layout_passes=False),
        out_type=jax.ShapeDtypeStruct((NV,), jnp.float32),
        scratch_types=[
            pltpu.VMEM((per_slab + _L,), jnp.float32),
            pltpu.VMEM((C,), jnp.int32),
            pltpu.VMEM((C,), jnp.float32),
            pltpu.VMEM((C,), jnp.int32),
            pltpu.VMEM((C,), jnp.float32),
            pltpu.VMEM((OVF,), jnp.int32),
            pltpu.VMEM((OVF,), jnp.float32),
        ],
    )
    def k(idx_hbm, val_hbm, g0_hbm, gout_hbm, slab, ib, vb, lcb, mb, ovi, ovv):
        lo = _wid() * per_slab
        dumpvec = per_slab + lax.iota(jnp.int32, _L)
        lane0 = lax.iota(jnp.int32, _L) == 0
        pltpu.sync_copy(
            g0_hbm.at[pl.ds(lo, per_slab)], slab.at[pl.ds(0, per_slab)]
        )

        def chunk(ci, carry):
            off = ci * C
            pltpu.sync_copy(idx_hbm.at[pl.ds(off, C)], ib)
            pltpu.sync_copy(val_hbm.at[pl.ds(off, C)], vb)

            def p1(i, c2):
                s = pl.ds(i * _L, _L)
                loc = ib[s] - lo
                msk = jnp.logical_and(loc >= 0, loc < per_slab)
                lc = jnp.where(msk, loc, dumpvec)
                g = plsc.load_gather(slab, [lc])
                mb[s] = jnp.maximum(g, vb[s])
                lcb[s] = lc
                return c2

            lax.fori_loop(0, C // _L, p1, 0, unroll=8)

            def p2(i, c2):
                s = pl.ds(i * _L, _L)
                plsc.store_scatter(slab, [lcb[s]], mb[s])
                return c2

            lax.fori_loop(0, C // _L, p2, 0, unroll=8)

            def p3(i, cnt):
                s = pl.ds(i * _L, _L)
                lc = lcb[s]
                vv = vb[s]
                g3 = plsc.load_gather(slab, [lc])
                bad = jnp.logical_and(g3 < vv, lc < per_slab)
                plsc.store_compressed(ovi.at[pl.ds(cnt, _L)], lc, mask=bad)
                plsc.store_compressed(ovv.at[pl.ds(cnt, _L)], vv, mask=bad)
                return cnt + plsc.all_reduce_population_count(bad)[0]

            cnt = lax.fori_loop(0, C // _L, p3, 0, unroll=8)

            def dbody(e):
                iivec = jnp.bitwise_and(ovi[pl.ds(e, _L)], per_slab - 1)
                uvec = ovv[pl.ds(e, _L)]
                g = plsc.load_gather(slab, [iivec])
                plsc.store_scatter(
                    slab, [iivec], jnp.maximum(g, uvec), mask=lane0
                )
                return e + 1

            lax.while_loop(lambda e: e < cnt, dbody, 0)
            return carry

        lax.fori_loop(0, NCH, chunk, 0)
        pltpu.sync_copy(
            slab.at[pl.ds(0, per_slab)], gout_hbm.at[pl.ds(lo, per_slab)]
        )

    return k


def _make_gather_kernel(N, NV):
    C = 8192
    per = N // _NW
    NCH = per // C

    @functools.partial(
        pl.kernel,
        mesh=_mesh(),
        compiler_params=pltpu.CompilerParams(needs_layout_passes=False),
        out_type=jax.ShapeDtypeStruct((N,), jnp.float32),
        scratch_types=[
            pltpu.VMEM((C,), jnp.int32),
            pltpu.VMEM((C,), jnp.float32),
            pltpu.SemaphoreType.DMA,
        ],
    )
    def k(g_hbm, idx_hbm, out_hbm, ib, ob, sem):
        base = _wid() * per

        def chunk(ci, carry):
            off = base + ci * C
            pltpu.sync_copy(idx_hbm.at[pl.ds(off, C)], ib)
            pltpu.async_copy(g_hbm.at[ib], ob, sem).wait()
            pltpu.sync_copy(ob, out_hbm.at[pl.ds(off, C)])
            return carry

        lax.fori_loop(0, NCH, chunk, 0)

    return k


def kernel(positions, new_levels, grid):
    N = positions.shape[0]
    res = grid.shape[0]
    NV = res * res * res

    x = positions[:, 0]
    y = positions[:, 1]
    z = positions[:, 2]

    idx = _make_idx_kernel(N, res)(x, y, z)
    gridf = grid.reshape(NV)
    g_final = _make_scatter_kernel(N, NV)(idx, new_levels, gridf)
    out = _make_gather_kernel(N, NV)(g_final, idx)
    return out.reshape(N, 1)


# speedup vs baseline: 2.2958x; 2.2958x over previous
import functools

import jax
import jax.numpy as jnp
from jax import lax
from jax.experimental import pallas as pl
from jax.experimental.pallas import tpu as pltpu
from jax.experimental.pallas import tpu_sc as plsc

_NC = 2
_NS = 16
_NW = _NC * _NS
_L = 16


def _mesh():
    return plsc.VectorSubcoreMesh(core_axis_name="c", subcore_axis_name="s")


def _wid():
    return lax.axis_index("s") * _NC + lax.axis_index("c")


def _make_idx_kernel(N, res):
    C = 16384
    per = N // _NW
    hi = jnp.float32(res - 1.001)
    scale = jnp.float32(res - 1)

    @functools.partial(
        pl.kernel,
        mesh=_mesh(),
        compiler_params=pltpu.CompilerParams(needs_layout_passes=False),
        out_type=jax.ShapeDtypeStruct((N,), jnp.int32),
        scratch_types=[
            pltpu.VMEM((C,), jnp.float32),
            pltpu.VMEM((C,), jnp.float32),
            pltpu.VMEM((C,), jnp.float32),
            pltpu.VMEM((C,), jnp.int32),
        ],
    )
    def k(x_hbm, y_hbm, z_hbm, idx_hbm, xb, yb, zb, ob):
        base = _wid() * per

        def chunk(ci, carry):
            off = base + ci * C
            pltpu.sync_copy(x_hbm.at[pl.ds(off, C)], xb)
            pltpu.sync_copy(y_hbm.at[pl.ds(off, C)], yb)
            pltpu.sync_copy(z_hbm.at[pl.ds(off, C)], zb)

            def vreg(i, c2):
                s = pl.ds(i * _L, _L)
                ix = jnp.clip(xb[s] * scale, 0.0, hi).astype(jnp.int32)
                iy = jnp.clip(yb[s] * scale, 0.0, hi).astype(jnp.int32)
                iz = jnp.clip(zb[s] * scale, 0.0, hi).astype(jnp.int32)
                ob[s] = ix * (res * res) + iy * res + iz
                return c2

            lax.fori_loop(0, C // _L, vreg, 0)
            pltpu.sync_copy(ob, idx_hbm.at[pl.ds(off, C)])
            return carry

        lax.fori_loop(0, per // C, chunk, 0)

    return k


def _make_scatter_kernel(N, NV):
    C = 8192
    NCH = N // C
    per_slab = NV // _NW

    STG = C + _L
    OVF = C + _L

    @functools.partial(
        pl.kernel,
        mesh=_mesh(),
        compiler_params=pltpu.CompilerParams(needs_layout_passes=False),
        out_type=jax.ShapeDtypeStruct((NV,), jnp.float32),
        scratch_types=[
            pltpu.VMEM((per_slab + _L,), jnp.float32),
            pltpu.VMEM((C,), jnp.int32),
            pltpu.VMEM((C,), jnp.float32),
            pltpu.VMEM((STG,), jnp.int32),
            pltpu.VMEM((STG,), jnp.float32),
            pltpu.VMEM((OVF,), jnp.int32),
            pltpu.VMEM((OVF,), jnp.float32),
        ],
    )
    def k(idx_hbm, val_hbm, g0_hbm, gout_hbm, slab, ib, vb, sti, stv, ovi, ovv):
        lo = _wid() * per_slab
        dumpvec = per_slab + lax.iota(jnp.int32, _L)
        lane0 = lax.iota(jnp.int32, _L) == 0
        pltpu.sync_copy(
            g0_hbm.at[pl.ds(lo, per_slab)], slab.at[pl.ds(0, per_slab)]
        )

        def chunk(ci, carry):
            off = ci * C
            pltpu.sync_copy(idx_hbm.at[pl.ds(off, C)], ib)
            pltpu.sync_copy(val_hbm.at[pl.ds(off, C)], vb)

            def pa(i, cnt):
                s = pl.ds(i * _L, _L)
                loc = ib[s] - lo
                vv = vb[s]
                msk = jnp.logical_and(loc >= 0, loc < per_slab)
                plsc.store_compressed(sti.at[pl.ds(cnt, _L)], loc, mask=msk)
                plsc.store_compressed(stv.at[pl.ds(cnt, _L)], vv, mask=msk)
                return cnt + plsc.all_reduce_population_count(msk)[0]

            cnt = lax.fori_loop(0, C // _L, pa, 0, unroll=8)

            sti[pl.ds(cnt, _L)] = dumpvec
            stv[pl.ds(cnt, _L)] = jnp.full((_L,), -1.0, jnp.float32)
            nb = cnt // _L + 1

            def pb(i, cnt2):
                s = pl.ds(i * _L, _L)
                lc = sti[s]
                vv = stv[s]
                g = plsc.load_gather(slab, [lc])
                plsc.store_scatter(slab, [lc], jnp.maximum(g, vv))
                g2 = plsc.load_gather(slab, [lc])
                bad = jnp.logical_and(g2 < vv, lc < per_slab)
                plsc.store_compressed(ovi.at[pl.ds(cnt2, _L)], lc, mask=bad)
                plsc.store_compressed(ovv.at[pl.ds(cnt2, _L)], vv, mask=bad)
                return cnt2 + plsc.all_reduce_population_count(bad)[0]

            cnt2 = lax.fori_loop(0, nb, pb, 0)

            def dbody(e):
                iivec = jnp.bitwise_and(ovi[pl.ds(e, _L)], per_slab - 1)
                uvec = ovv[pl.ds(e, _L)]
                g = plsc.load_gather(slab, [iivec])
                plsc.store_scatter(
                    slab, [iivec], jnp.maximum(g, uvec), mask=lane0
                )
                return e + 1

            lax.while_loop(lambda e: e < cnt2, dbody, 0)
            return carry

        lax.fori_loop(0, NCH, chunk, 0)
        pltpu.sync_copy(
            slab.at[pl.ds(0, per_slab)], gout_hbm.at[pl.ds(lo, per_slab)]
        )

    return k


def _make_gather_kernel(N, NV):
    C = 8192
    per = N // _NW
    NCH = per // C

    @functools.partial(
        pl.kernel,
        mesh=_mesh(),
        compiler_params=pltpu.CompilerParams(needs_layout_passes=False),
        out_type=jax.ShapeDtypeStruct((N,), jnp.float32),
        scratch_types=[
            pltpu.VMEM((C,), jnp.int32),
            pltpu.VMEM((C,), jnp.float32),
            pltpu.SemaphoreType.DMA,
        ],
    )
    def k(g_hbm, idx_hbm, out_hbm, ib, ob, sem):
        base = _wid() * per

        def chunk(ci, carry):
            off = base + ci * C
            pltpu.sync_copy(idx_hbm.at[pl.ds(off, C)], ib)
            pltpu.async_copy(g_hbm.at[ib], ob, sem).wait()
            pltpu.sync_copy(ob, out_hbm.at[pl.ds(off, C)])
            return carry

        lax.fori_loop(0, NCH, chunk, 0)

    return k


def kernel(positions, new_levels, grid):
    N = positions.shape[0]
    res = grid.shape[0]
    NV = res * res * res

    x = positions[:, 0]
    y = positions[:, 1]
    z = positions[:, 2]

    idx = _make_idx_kernel(N, res)(x, y, z)
    gridf = grid.reshape(NV)
    g_final = _make_scatter_kernel(N, NV)(idx, new_levels, gridf)
    out = _make_gather_kernel(N, NV)(g_final, idx)
    return out.reshape(N, 1)


# speedup vs baseline: 2.9939x; 1.3041x over previous
import functools

import jax
import jax.numpy as jnp
from jax import lax
from jax.experimental import pallas as pl
from jax.experimental.pallas import tpu as pltpu
from jax.experimental.pallas import tpu_sc as plsc

_NC = 2
_NS = 16
_NW = _NC * _NS
_L = 16


def _mesh():
    return plsc.VectorSubcoreMesh(core_axis_name="c", subcore_axis_name="s")


def _wid():
    return lax.axis_index("s") * _NC + lax.axis_index("c")


def _make_idx_kernel(N, res):
    C = 16384
    per = N // _NW
    hi = jnp.float32(res - 1.001)
    scale = jnp.float32(res - 1)

    @functools.partial(
        pl.kernel,
        mesh=_mesh(),
        compiler_params=pltpu.CompilerParams(needs_layout_passes=False),
        out_type=jax.ShapeDtypeStruct((N,), jnp.int32),
        scratch_types=[
            pltpu.VMEM((C,), jnp.float32),
            pltpu.VMEM((C,), jnp.float32),
            pltpu.VMEM((C,), jnp.float32),
            pltpu.VMEM((C,), jnp.int32),
        ],
    )
    def k(x_hbm, y_hbm, z_hbm, idx_hbm, xb, yb, zb, ob):
        base = _wid() * per

        def chunk(ci, carry):
            off = base + ci * C
            pltpu.sync_copy(x_hbm.at[pl.ds(off, C)], xb)
            pltpu.sync_copy(y_hbm.at[pl.ds(off, C)], yb)
            pltpu.sync_copy(z_hbm.at[pl.ds(off, C)], zb)

            def vreg(i, c2):
                s = pl.ds(i * _L, _L)
                ix = jnp.clip(xb[s] * scale, 0.0, hi).astype(jnp.int32)
                iy = jnp.clip(yb[s] * scale, 0.0, hi).astype(jnp.int32)
                iz = jnp.clip(zb[s] * scale, 0.0, hi).astype(jnp.int32)
                ob[s] = ix * (res * res) + iy * res + iz
                return c2

            lax.fori_loop(0, C // _L, vreg, 0)
            pltpu.sync_copy(ob, idx_hbm.at[pl.ds(off, C)])
            return carry

        lax.fori_loop(0, per // C, chunk, 0)

    return k


def _make_scatter_kernel(N, NV):
    C = 4096
    NCH = N // C
    NP = NCH // 2
    per_slab = NV // _NW

    STG = C + _L
    OVF = C + _L

    @functools.partial(
        pl.kernel,
        mesh=_mesh(),
        compiler_params=pltpu.CompilerParams(needs_layout_passes=False),
        out_type=jax.ShapeDtypeStruct((NV,), jnp.float32),
        scratch_types=[
            pltpu.VMEM((per_slab + _L,), jnp.float32),
            pltpu.VMEM((C,), jnp.int32),
            pltpu.VMEM((C,), jnp.float32),
            pltpu.VMEM((C,), jnp.int32),
            pltpu.VMEM((C,), jnp.float32),
            pltpu.VMEM((STG,), jnp.int32),
            pltpu.VMEM((STG,), jnp.float32),
            pltpu.VMEM((OVF,), jnp.int32),
            pltpu.VMEM((OVF,), jnp.float32),
            pltpu.SemaphoreType.DMA,
            pltpu.SemaphoreType.DMA,
        ],
    )
    def k(idx_hbm, val_hbm, g0_hbm, gout_hbm, slab,
          iba, vba, ibb, vbb, sti, stv, ovi, ovv, sema, semb):
        lo = _wid() * per_slab
        dumpvec = per_slab + lax.iota(jnp.int32, _L)
        lane0 = lax.iota(jnp.int32, _L) == 0
        pltpu.sync_copy(
            g0_hbm.at[pl.ds(lo, per_slab)], slab.at[pl.ds(0, per_slab)]
        )

        def issue(ci, ib, vb, sem):
            pltpu.async_copy(idx_hbm.at[pl.ds(ci * C, C)], ib, sem)
            pltpu.async_copy(val_hbm.at[pl.ds(ci * C, C)], vb, sem)

        def drain_dma(ib, vb, sem):
            pltpu.make_async_copy(idx_hbm.at[pl.ds(0, C)], ib, sem).wait()
            pltpu.make_async_copy(val_hbm.at[pl.ds(0, C)], vb, sem).wait()

        def compute(ib, vb):
            def pa(i, cnt):
                s = pl.ds(i * _L, _L)
                loc = ib[s] - lo
                vv = vb[s]
                msk = jnp.logical_and(loc >= 0, loc < per_slab)
                plsc.store_compressed(sti.at[pl.ds(cnt, _L)], loc, mask=msk)
                plsc.store_compressed(stv.at[pl.ds(cnt, _L)], vv, mask=msk)
                return cnt + plsc.all_reduce_population_count(msk)[0]

            cnt = lax.fori_loop(0, C // _L, pa, 0, unroll=8)

            sti[pl.ds(cnt, _L)] = dumpvec
            stv[pl.ds(cnt, _L)] = jnp.full((_L,), -1.0, jnp.float32)
            nb = cnt // _L + 1

            def pb(i, cnt2):
                s = pl.ds(i * _L, _L)
                lc = sti[s]
                vv = stv[s]
                g = plsc.load_gather(slab, [lc])
                plsc.store_scatter(slab, [lc], jnp.maximum(g, vv))
                g2 = plsc.load_gather(slab, [lc])
                bad = jnp.logical_and(g2 < vv, lc < per_slab)
                plsc.store_compressed(ovi.at[pl.ds(cnt2, _L)], lc, mask=bad)
                plsc.store_compressed(ovv.at[pl.ds(cnt2, _L)], vv, mask=bad)
                return cnt2 + plsc.all_reduce_population_count(bad)[0]

            cnt2 = lax.fori_loop(0, nb, pb, 0)

            def dbody(e):
                iivec = jnp.bitwise_and(ovi[pl.ds(e, _L)], per_slab - 1)
                uvec = ovv[pl.ds(e, _L)]
                g = plsc.load_gather(slab, [iivec])
                plsc.store_scatter(
                    slab, [iivec], jnp.maximum(g, uvec), mask=lane0
                )
                return e + 1

            lax.while_loop(lambda e: e < cnt2, dbody, 0)

        issue(0, iba, vba, sema)

        def pair(kk, carry):
            issue(2 * kk + 1, ibb, vbb, semb)
            drain_dma(iba, vba, sema)
            compute(iba, vba)
            issue(2 * kk + 2, iba, vba, sema)
            drain_dma(ibb, vbb, semb)
            compute(ibb, vbb)
            return carry

        lax.fori_loop(0, NP - 1, pair, 0)
        issue(NCH - 1, ibb, vbb, semb)
        drain_dma(iba, vba, sema)
        compute(iba, vba)
        drain_dma(ibb, vbb, semb)
        compute(ibb, vbb)

        pltpu.sync_copy(
            slab.at[pl.ds(0, per_slab)], gout_hbm.at[pl.ds(lo, per_slab)]
        )

    return k


def _make_gather_kernel(N, NV):
    C = 8192
    per = N // _NW
    NCH = per // C

    @functools.partial(
        pl.kernel,
        mesh=_mesh(),
        compiler_params=pltpu.CompilerParams(needs_layout_passes=False),
        out_type=jax.ShapeDtypeStruct((N,), jnp.float32),
        scratch_types=[
            pltpu.VMEM((C,), jnp.int32),
            pltpu.VMEM((C,), jnp.float32),
            pltpu.SemaphoreType.DMA,
        ],
    )
    def k(g_hbm, idx_hbm, out_hbm, ib, ob, sem):
        base = _wid() * per

        def chunk(ci, carry):
            off = base + ci * C
            pltpu.sync_copy(idx_hbm.at[pl.ds(off, C)], ib)
            pltpu.async_copy(g_hbm.at[ib], ob, sem).wait()
            pltpu.sync_copy(ob, out_hbm.at[pl.ds(off, C)])
            return carry

        lax.fori_loop(0, NCH, chunk, 0)

    return k


def kernel(positions, new_levels, grid):
    N = positions.shape[0]
    res = grid.shape[0]
    NV = res * res * res

    x = positions[:, 0]
    y = positions[:, 1]
    z = positions[:, 2]

    idx = _make_idx_kernel(N, res)(x, y, z)
    gridf = grid.reshape(NV)
    g_final = _make_scatter_kernel(N, NV)(idx, new_levels, gridf)
    out = _make_gather_kernel(N, NV)(g_final, idx)
    return out.reshape(N, 1)
